# Initial kernel scaffold; baseline (speedup 1.0000x reference)
import jax
import jax.numpy as jnp
from jax.experimental import pallas as pl


def kernel(x, edge_index, batch, W1, b1, W2, b2, W3, b3, Wl, bl):
    raise NotImplementedError("write your pallas kernel here")


# speedup vs baseline: 13.4364x; 13.4364x over previous
import functools

import jax
import jax.numpy as jnp
from jax import lax
from jax.experimental import pallas as pl
from jax.experimental.pallas import tpu as pltpu
from jax.experimental.pallas import tpu_sc as plsc

N = 10000
NP = 10240
E = 320000
D = 128
G = 64
NC, NS = 2, 16
NW = NC * NS
EPT = E // NW
CH = 80
NCHUNK = EPT // CH
RPT = NP // NS
ZR = 128

_mesh = plsc.VectorSubcoreMesh(
    core_axis_name="c", subcore_axis_name="s", num_cores=NC, num_subcores=NS)


def _zero_vmem_2d(ref, rows, cols):
  def row(i, _):
    for j in range(cols // 16):
      ref[i, pl.ds(j * 16, 16)] = jnp.zeros((16,), jnp.float32)
    return _
  lax.fori_loop(0, rows, row, 0)


@functools.partial(
    pl.kernel,
    out_type=jax.ShapeDtypeStruct((NC, NP, 16), jnp.float32),
    mesh=_mesh,
    scratch_types=[
        pltpu.VMEM((1, CH), jnp.int32),
        pltpu.VMEM((CH, 16), jnp.float32),
        pltpu.VMEM((ZR, 16), jnp.float32),
        pltpu.VMEM_SHARED((NP, 16), jnp.float32),
    ],
)
def sc_degree(dst_hbm, out_hbm, dbuf, ones_b, zbuf, acc):
  c = lax.axis_index("c")
  s = lax.axis_index("s")
  ebase = (c * NS + s) * EPT
  row0 = s * RPT

  _zero_vmem_2d(zbuf, ZR, 16)
  def fill(i, _):
    ones_b[i, pl.ds(0, 16)] = jnp.ones((16,), jnp.float32)
    return _
  lax.fori_loop(0, CH, fill, 0)
  for k in range(RPT // ZR):
    pltpu.sync_copy(zbuf, acc.at[pl.ds(row0 + k * ZR, ZR)])
  plsc.subcore_barrier()

  def step(j, _):
    pltpu.sync_copy(dst_hbm.at[pl.ds(ebase + j * CH, CH)], dbuf.at[0])
    pltpu.sync_copy(ones_b, acc.at[dbuf.at[0]], add=True)
    return _
  lax.fori_loop(0, NCHUNK, step, 0)

  plsc.subcore_barrier()
  for k in range(RPT // ZR):
    r = row0 + k * ZR
    pltpu.sync_copy(acc.at[pl.ds(r, ZR)], out_hbm.at[c, pl.ds(r, ZR)])


@functools.partial(
    pl.kernel,
    out_type=jax.ShapeDtypeStruct((NC, NP, D), jnp.float32),
    mesh=_mesh,
    scratch_types=[
        pltpu.VMEM((1, CH), jnp.int32),
        pltpu.VMEM((1, CH), jnp.int32),
        pltpu.VMEM((1, CH), jnp.int32),
        pltpu.VMEM((1, CH), jnp.int32),
        pltpu.VMEM((CH, D), jnp.float32),
        pltpu.VMEM((CH, D), jnp.float32),
        pltpu.VMEM((ZR, D), jnp.float32),
        pltpu.VMEM_SHARED((NP, D), jnp.float32),
        pltpu.SemaphoreType.DMA,
        pltpu.SemaphoreType.DMA,
    ],
)
def sc_propagate(g_hbm, src_hbm, dst_hbm, out_hbm,
                 sb0, sb1, db0, db1, rb0, rb1, zbuf, acc, sem0, sem1):
  c = lax.axis_index("c")
  s = lax.axis_index("s")
  ebase = (c * NS + s) * EPT
  row0 = s * RPT

  _zero_vmem_2d(zbuf, ZR, D)
  for k in range(RPT // ZR):
    pltpu.sync_copy(zbuf, acc.at[pl.ds(row0 + k * ZR, ZR)])
  plsc.subcore_barrier()

  sbufs = (sb0, sb1)
  dbufs = (db0, db1)
  rbufs = (rb0, rb1)
  sems = (sem0, sem1)

  def prefetch(j, slot):
    pltpu.sync_copy(src_hbm.at[pl.ds(ebase + j * CH, CH)], sbufs[slot].at[0])
    pltpu.sync_copy(dst_hbm.at[pl.ds(ebase + j * CH, CH)], dbufs[slot].at[0])
    pltpu.async_copy(g_hbm.at[sbufs[slot].at[0]], rbufs[slot], sems[slot])

  prefetch(0, 0)

  def step(j, carry):
    slot = lax.rem(j, 2)
    def body(sl):
      pltpu.make_async_copy(g_hbm.at[sbufs[sl].at[0]], rbufs[sl],
                            sems[sl]).wait()
      @pl.when(j + 1 < NCHUNK)
      def _pref():
        prefetch(j + 1, 1 - sl)
      pltpu.sync_copy(rbufs[sl], acc.at[dbufs[sl].at[0]], add=True)
    @pl.when(slot == 0)
    def _s0():
      body(0)
    @pl.when(slot == 1)
    def _s1():
      body(1)
    return carry
  lax.fori_loop(0, NCHUNK, step, 0)

  plsc.subcore_barrier()
  for k in range(RPT // ZR):
    r = row0 + k * ZR
    pltpu.sync_copy(acc.at[pl.ds(r, ZR)], out_hbm.at[c, pl.ds(r, ZR)])


def _tc_first(x_ref, w_ref, degp_ref, g_ref, dinv_ref):
  deg = 1.0 + degp_ref[0][:, 0:1] + degp_ref[1][:, 0:1]
  dinv = jnp.broadcast_to(lax.rsqrt(deg), (NP, D))
  dinv_ref[...] = dinv
  g_ref[...] = dinv * jnp.dot(x_ref[...], w_ref[...],
                              preferred_element_type=jnp.float32)


def _tc_mid(y_ref, g_ref, dinv_ref, b_ref, w_ref, gout_ref):
  dinv = dinv_ref[...]
  h = jnp.maximum(dinv * (y_ref[0] + y_ref[1] + g_ref[...]) + b_ref[...], 0.0)
  gout_ref[...] = dinv * jnp.dot(h, w_ref[...],
                                 preferred_element_type=jnp.float32)


def _tc_last(y_ref, g_ref, dinv_ref, b_ref, batch_ref, wl_ref, bl_ref,
             out_ref):
  dinv = dinv_ref[...]
  h = dinv * (y_ref[0] + y_ref[1] + g_ref[...]) + b_ref[...]
  ids = lax.broadcasted_iota(jnp.int32, (G, NP), 0)
  ind = (ids == jnp.broadcast_to(batch_ref[...], (G, NP))).astype(jnp.float32)
  sums = jnp.dot(ind, h, preferred_element_type=jnp.float32)
  cnts = jnp.sum(ind, axis=1, keepdims=True)
  pooled = sums / jnp.maximum(cnts, 1.0)
  logits = jnp.dot(pooled, wl_ref[...],
                   preferred_element_type=jnp.float32) + bl_ref[...]
  col = lax.broadcasted_iota(jnp.int32, (G, D), 1)
  lm = jnp.where(col < 10, logits, -jnp.inf)
  m = jnp.max(lm, axis=1, keepdims=True)
  lse = m + jnp.log(jnp.sum(jnp.exp(lm - m), axis=1, keepdims=True))
  out_ref[...] = logits - lse


def kernel(x, edge_index, batch, W1, b1, W2, b2, W3, b3, Wl, bl):
  src = edge_index[0].astype(jnp.int32)
  dst = edge_index[1].astype(jnp.int32)
  x = jnp.pad(x, ((0, NP - N), (0, 0)))
  batch2 = jnp.pad(batch.astype(jnp.int32), (0, NP - N),
                   constant_values=G).reshape(1, NP)
  wl_pad = jnp.zeros((D, D), jnp.float32).at[:, :10].set(Wl)
  bl_pad = jnp.zeros((1, D), jnp.float32).at[:, :10].set(bl)

  degp = sc_degree(dst)

  g1, dinv = pl.pallas_call(
      _tc_first,
      out_shape=(jax.ShapeDtypeStruct((NP, D), jnp.float32),
                 jax.ShapeDtypeStruct((NP, D), jnp.float32)),
  )(x, W1, degp)

  y1 = sc_propagate(g1, src, dst)

  mid = pl.pallas_call(
      _tc_mid, out_shape=jax.ShapeDtypeStruct((NP, D), jnp.float32))
  g2 = mid(y1, g1, dinv, b1.reshape(1, D), W2)
  y2 = sc_propagate(g2, src, dst)
  g3 = mid(y2, g2, dinv, b2.reshape(1, D), W3)
  y3 = sc_propagate(g3, src, dst)

  out = pl.pallas_call(
      _tc_last, out_shape=jax.ShapeDtypeStruct((G, D), jnp.float32))(
          y3, g3, dinv, b3.reshape(1, D), batch2, wl_pad, bl_pad)
  return out[:, :10]


# speedup vs baseline: 26.4950x; 1.9719x over previous
import functools

import jax
import jax.numpy as jnp
from jax import lax
from jax.experimental import pallas as pl
from jax.experimental.pallas import tpu as pltpu
from jax.experimental.pallas import tpu_sc as plsc

N = 10000
NP = 10240
E = 320000
D = 128
G = 64
NC, NS = 2, 16
NW = NC * NS
EPT = E // NW
CH = 40
NCHUNK = EPT // CH
RPT = NP // NS
ZR = 128
NBUF = 5
GDEPTH = 3

_mesh = plsc.VectorSubcoreMesh(
    core_axis_name="c", subcore_axis_name="s", num_cores=NC, num_subcores=NS)


def _zero_vmem_2d(ref, rows, cols):
  def row(i, _):
    for j in range(cols // 16):
      ref[i, pl.ds(j * 16, 16)] = jnp.zeros((16,), jnp.float32)
    return _
  lax.fori_loop(0, rows, row, 0)


@functools.partial(
    pl.kernel,
    out_type=jax.ShapeDtypeStruct((NC, NP, D), jnp.float32),
    mesh=_mesh,
    scratch_types=[
        pltpu.VMEM((1, CH), jnp.int32),
        pltpu.VMEM((1, CH), jnp.int32),
        pltpu.VMEM((1, CH), jnp.int32),
        pltpu.VMEM((1, CH), jnp.int32),
        pltpu.VMEM((1, CH), jnp.int32),
        pltpu.VMEM((CH, D), jnp.float32),
        pltpu.VMEM((CH, D), jnp.float32),
        pltpu.VMEM_SHARED((NP, D), jnp.float32),
        pltpu.SemaphoreType.DMA,
        pltpu.SemaphoreType.DMA,
        pltpu.SemaphoreType.DMA,
        pltpu.SemaphoreType.DMA,
        pltpu.SemaphoreType.DMA,
        pltpu.SemaphoreType.DMA,
        pltpu.SemaphoreType.DMA,
        pltpu.SemaphoreType.DMA,
        pltpu.SemaphoreType.DMA,
        pltpu.SemaphoreType.DMA,
        pltpu.SemaphoreType.DMA,
    ],
)
def sc_degree(dst_hbm, out_hbm, ib0, ib1, ib2, ib3, ib4, ones_b, zbuf, acc,
              si0, si1, si2, si3, si4, ss0, ss1, ss2, ss3, ss4, sb):
  c = lax.axis_index("c")
  s = lax.axis_index("s")
  ebase = (c * NS + s) * EPT
  row0 = s * RPT
  ibufs = (ib0, ib1, ib2, ib3, ib4)
  si = (si0, si1, si2, si3, si4)
  ss = (ss0, ss1, ss2, ss3, ss4)

  _zero_vmem_2d(zbuf, CH, D)
  def fill(i, u):
    for jj in range(D // 16):
      ones_b[i, pl.ds(jj * 16, 16)] = jnp.ones((16,), jnp.float32)
    return u
  lax.fori_loop(0, CH, fill, 0)
  for k in range(RPT // CH):
    pltpu.async_copy(zbuf, acc.at[pl.ds(row0 + k * CH, CH)], sb)
  for k in range(RPT // CH):
    pltpu.make_async_copy(zbuf, acc.at[pl.ds(row0 + k * CH, CH)], sb).wait()
  plsc.subcore_barrier()

  def iissue(j, b):
    pltpu.async_copy(dst_hbm.at[pl.ds(ebase + j * CH, CH)], ibufs[b].at[0],
                     si[b])

  def iwait(j, b):
    pltpu.make_async_copy(dst_hbm.at[pl.ds(ebase + j * CH, CH)],
                          ibufs[b].at[0], si[b]).wait()

  def sissue(j, b):
    pltpu.async_copy(ones_b, acc.at[ibufs[b].at[0]], ss[b], add=True)

  def swait(j, b):
    pltpu.make_async_copy(ones_b, acc.at[ibufs[b].at[0]], ss[b]).wait()

  for b in range(2):
    iissue(b, b)

  def outer(kk, carry):
    for b in range(NBUF):
      j = kk * NBUF + b
      nb = (b + 2) % NBUF
      @pl.when(j >= 3)
      def _ws():
        swait(j - 3, nb)
      @pl.when(j + 2 < NCHUNK)
      def _ii():
        iissue(j + 2, nb)
      iwait(j, b)
      sissue(j, b)
    return carry
  lax.fori_loop(0, NCHUNK // NBUF, outer, 0)
  for t in range(3):
    jj = NCHUNK - 3 + t
    swait(jj, jj % NBUF)

  plsc.subcore_barrier()
  for k in range(RPT // CH):
    r = row0 + k * CH
    pltpu.async_copy(acc.at[pl.ds(r, CH)], out_hbm.at[c, pl.ds(r, CH)], sb)
  for k in range(RPT // CH):
    r = row0 + k * CH
    pltpu.make_async_copy(acc.at[pl.ds(r, CH)], out_hbm.at[c, pl.ds(r, CH)],
                          sb).wait()


@functools.partial(
    pl.kernel,
    out_type=jax.ShapeDtypeStruct((NC, NP, D), jnp.float32),
    mesh=_mesh,
    scratch_types=[
        pltpu.VMEM((2, CH), jnp.int32),
        pltpu.VMEM((2, CH), jnp.int32),
        pltpu.VMEM((2, CH), jnp.int32),
        pltpu.VMEM((2, CH), jnp.int32),
        pltpu.VMEM((2, CH), jnp.int32),
        pltpu.VMEM((CH, D), jnp.float32),
        pltpu.VMEM((CH, D), jnp.float32),
        pltpu.VMEM((CH, D), jnp.float32),
        pltpu.VMEM((CH, D), jnp.float32),
        pltpu.VMEM((CH, D), jnp.float32),
        pltpu.VMEM_SHARED((NP, D), jnp.float32),
        pltpu.SemaphoreType.DMA,
        pltpu.SemaphoreType.DMA,
        pltpu.SemaphoreType.DMA,
        pltpu.SemaphoreType.DMA,
        pltpu.SemaphoreType.DMA,
        pltpu.SemaphoreType.DMA,
        pltpu.SemaphoreType.DMA,
        pltpu.SemaphoreType.DMA,
        pltpu.SemaphoreType.DMA,
        pltpu.SemaphoreType.DMA,
        pltpu.SemaphoreType.DMA,
        pltpu.SemaphoreType.DMA,
        pltpu.SemaphoreType.DMA,
        pltpu.SemaphoreType.DMA,
        pltpu.SemaphoreType.DMA,
        pltpu.SemaphoreType.DMA,
    ],
)
def sc_propagate(g_hbm, src_hbm, dst_hbm, out_hbm,
                 ib0, ib1, ib2, ib3, ib4, rb0, rb1, rb2, rb3, rb4, acc,
                 si0, si1, si2, si3, si4, sg0, sg1, sg2, sg3, sg4,
                 ss0, ss1, ss2, ss3, ss4, sb):
  c = lax.axis_index("c")
  s = lax.axis_index("s")
  ebase = (c * NS + s) * EPT
  row0 = s * RPT
  ibufs = (ib0, ib1, ib2, ib3, ib4)
  rbufs = (rb0, rb1, rb2, rb3, rb4)
  si = (si0, si1, si2, si3, si4)
  sg = (sg0, sg1, sg2, sg3, sg4)
  ss = (ss0, ss1, ss2, ss3, ss4)

  _zero_vmem_2d(rb0, CH, D)
  for k in range(RPT // CH):
    pltpu.async_copy(rb0, acc.at[pl.ds(row0 + k * CH, CH)], sb)
  for k in range(RPT // CH):
    pltpu.make_async_copy(rb0, acc.at[pl.ds(row0 + k * CH, CH)], sb).wait()
  plsc.subcore_barrier()

  def iissue(j, b):
    pltpu.async_copy(src_hbm.at[pl.ds(ebase + j * CH, CH)], ibufs[b].at[0],
                     si[b])
    pltpu.async_copy(dst_hbm.at[pl.ds(ebase + j * CH, CH)], ibufs[b].at[1],
                     si[b])

  def iwait(j, b):
    pltpu.make_async_copy(src_hbm.at[pl.ds(ebase + j * CH, CH)],
                          ibufs[b].at[0], si[b]).wait()
    pltpu.make_async_copy(dst_hbm.at[pl.ds(ebase + j * CH, CH)],
                          ibufs[b].at[1], si[b]).wait()

  def gissue(j, b):
    pltpu.async_copy(g_hbm.at[ibufs[b].at[0]], rbufs[b], sg[b])

  def gwait(j, b):
    pltpu.make_async_copy(g_hbm.at[ibufs[b].at[0]], rbufs[b], sg[b]).wait()

  def sissue(j, b):
    pltpu.async_copy(rbufs[b], acc.at[ibufs[b].at[1]], ss[b], add=True)

  def swait(j, b):
    pltpu.make_async_copy(rbufs[b], acc.at[ibufs[b].at[1]], ss[b]).wait()

  for b in range(3):
    iissue(b, b)
  for b in range(2):
    iwait(b, b)
    gissue(b, b)

  def outer(kk, carry):
    for b in range(NBUF):
      j = kk * NBUF + b
      nb3 = (b + 3) % NBUF
      nb2 = (b + 2) % NBUF
      @pl.when(j >= 2)
      def _ws():
        swait(j - 2, nb3)
      @pl.when(j + 3 < NCHUNK)
      def _ii():
        iissue(j + 3, nb3)
      @pl.when(j + 2 < NCHUNK)
      def _ig():
        iwait(j + 2, nb2)
        gissue(j + 2, nb2)
      gwait(j, b)
      sissue(j, b)
    return carry
  lax.fori_loop(0, NCHUNK // NBUF, outer, 0)
  for t in range(2):
    jj = NCHUNK - 2 + t
    swait(jj, jj % NBUF)

  plsc.subcore_barrier()
  for k in range(RPT // CH):
    r = row0 + k * CH
    pltpu.async_copy(acc.at[pl.ds(r, CH)], out_hbm.at[c, pl.ds(r, CH)], sb)
  for k in range(RPT // CH):
    r = row0 + k * CH
    pltpu.make_async_copy(acc.at[pl.ds(r, CH)], out_hbm.at[c, pl.ds(r, CH)],
                          sb).wait()


def _tc_first(x_ref, w_ref, degp_ref, g_ref, dinv_ref):
  deg = 1.0 + degp_ref[0][:, 0:1] + degp_ref[1][:, 0:1]
  dinv = jnp.broadcast_to(lax.rsqrt(deg), (NP, D))
  dinv_ref[...] = dinv
  g_ref[...] = dinv * jnp.dot(x_ref[...], w_ref[...],
                              preferred_element_type=jnp.float32)


def _tc_mid(y_ref, g_ref, dinv_ref, b_ref, w_ref, gout_ref):
  dinv = dinv_ref[...]
  h = jnp.maximum(dinv * (y_ref[0] + y_ref[1] + g_ref[...]) + b_ref[...], 0.0)
  gout_ref[...] = dinv * jnp.dot(h, w_ref[...],
                                 preferred_element_type=jnp.float32)


def _tc_last(y_ref, g_ref, dinv_ref, b_ref, batch_ref, wl_ref, bl_ref,
             out_ref):
  dinv = dinv_ref[...]
  h = dinv * (y_ref[0] + y_ref[1] + g_ref[...]) + b_ref[...]
  ids = lax.broadcasted_iota(jnp.int32, (G, NP), 0)
  ind = (ids == jnp.broadcast_to(batch_ref[...], (G, NP))).astype(jnp.float32)
  sums = jnp.dot(ind, h, preferred_element_type=jnp.float32)
  cnts = jnp.sum(ind, axis=1, keepdims=True)
  pooled = sums / jnp.maximum(cnts, 1.0)
  logits = jnp.dot(pooled, wl_ref[...],
                   preferred_element_type=jnp.float32) + bl_ref[...]
  col = lax.broadcasted_iota(jnp.int32, (G, D), 1)
  lm = jnp.where(col < 10, logits, -jnp.inf)
  m = jnp.max(lm, axis=1, keepdims=True)
  lse = m + jnp.log(jnp.sum(jnp.exp(lm - m), axis=1, keepdims=True))
  out_ref[...] = logits - lse


def kernel(x, edge_index, batch, W1, b1, W2, b2, W3, b3, Wl, bl):
  src = edge_index[0].astype(jnp.int32)
  dst = edge_index[1].astype(jnp.int32)
  x = jnp.pad(x, ((0, NP - N), (0, 0)))
  batch2 = jnp.pad(batch.astype(jnp.int32), (0, NP - N),
                   constant_values=G).reshape(1, NP)
  wl_pad = jnp.zeros((D, D), jnp.float32).at[:, :10].set(Wl)
  bl_pad = jnp.zeros((1, D), jnp.float32).at[:, :10].set(bl)

  degp = sc_degree(dst)

  g1, dinv = pl.pallas_call(
      _tc_first,
      out_shape=(jax.ShapeDtypeStruct((NP, D), jnp.float32),
                 jax.ShapeDtypeStruct((NP, D), jnp.float32)),
  )(x, W1, degp)

  y1 = sc_propagate(g1, src, dst)

  mid = pl.pallas_call(
      _tc_mid, out_shape=jax.ShapeDtypeStruct((NP, D), jnp.float32))
  g2 = mid(y1, g1, dinv, b1.reshape(1, D), W2)
  y2 = sc_propagate(g2, src, dst)
  g3 = mid(y2, g2, dinv, b2.reshape(1, D), W3)
  y3 = sc_propagate(g3, src, dst)

  out = pl.pallas_call(
      _tc_last, out_shape=jax.ShapeDtypeStruct((G, D), jnp.float32))(
          y3, g3, dinv, b3.reshape(1, D), batch2, wl_pad, bl_pad)
  return out[:, :10]


# speedup vs baseline: 30.3033x; 1.1437x over previous
import functools

import jax
import jax.numpy as jnp
from jax import lax
from jax.experimental import pallas as pl
from jax.experimental.pallas import tpu as pltpu
from jax.experimental.pallas import tpu_sc as plsc

N = 10000
NP = 10240
E = 320000
D = 128
G = 64
NC, NS = 2, 16
NW = NC * NS
EPT = E // NW
CH = 40
NCHUNK = EPT // CH
RPT = NP // NS
ZR = 128
NBUF = 5
GDEPTH = 3

_mesh = plsc.VectorSubcoreMesh(
    core_axis_name="c", subcore_axis_name="s", num_cores=NC, num_subcores=NS)


def _zero_vmem_2d(ref, rows, cols):
  def row(i, _):
    for j in range(cols // 16):
      ref[i, pl.ds(j * 16, 16)] = jnp.zeros((16,), jnp.float32)
    return _
  lax.fori_loop(0, rows, row, 0)


HR = NP // 128


@functools.partial(
    pl.kernel,
    out_type=jax.ShapeDtypeStruct((NW, HR, 128), jnp.float32),
    mesh=_mesh,
    compiler_params=pltpu.CompilerParams(needs_layout_passes=False),
    scratch_types=[
        pltpu.VMEM((1, EPT), jnp.int32),
        pltpu.VMEM((HR, 128), jnp.float32),
        pltpu.SemaphoreType.DMA,
    ],
)
def sc_degree(dst_hbm, out_hbm, ib, hist, sb):
  c = lax.axis_index("c")
  s = lax.axis_index("s")
  wid = c * NS + s
  ebase = wid * EPT

  pltpu.async_copy(dst_hbm.at[pl.ds(ebase, EPT)], ib.at[0], sb)
  def z(i, u):
    for j8 in range(8):
      hist[i, pl.ds(j8 * 16, 16)] = jnp.zeros((16,), jnp.float32)
    return u
  lax.fori_loop(0, HR, z, 0)
  pltpu.make_async_copy(dst_hbm.at[pl.ds(ebase, EPT)], ib.at[0], sb).wait()

  ones = jnp.ones((16,), jnp.float32)
  def st(i, u):
    v = ib[0, pl.ds(i * 16, 16)]
    plsc.addupdate_scatter(
        hist, [lax.shift_right_logical(v, 7), lax.bitwise_and(v, 127)], ones)
    return u
  lax.fori_loop(0, EPT // 16, st, 0)
  pltpu.sync_copy(hist, out_hbm.at[wid])


@functools.partial(
    pl.kernel,
    out_type=jax.ShapeDtypeStruct((NC, NP, D), jnp.float32),
    mesh=_mesh,
    scratch_types=[
        pltpu.VMEM((2, CH), jnp.int32),
        pltpu.VMEM((2, CH), jnp.int32),
        pltpu.VMEM((2, CH), jnp.int32),
        pltpu.VMEM((2, CH), jnp.int32),
        pltpu.VMEM((2, CH), jnp.int32),
        pltpu.VMEM((CH, D), jnp.float32),
        pltpu.VMEM((CH, D), jnp.float32),
        pltpu.VMEM((CH, D), jnp.float32),
        pltpu.VMEM((CH, D), jnp.float32),
        pltpu.VMEM((CH, D), jnp.float32),
        pltpu.VMEM_SHARED((NP, D), jnp.float32),
        pltpu.SemaphoreType.DMA,
        pltpu.SemaphoreType.DMA,
        pltpu.SemaphoreType.DMA,
        pltpu.SemaphoreType.DMA,
        pltpu.SemaphoreType.DMA,
        pltpu.SemaphoreType.DMA,
        pltpu.SemaphoreType.DMA,
        pltpu.SemaphoreType.DMA,
        pltpu.SemaphoreType.DMA,
        pltpu.SemaphoreType.DMA,
        pltpu.SemaphoreType.DMA,
        pltpu.SemaphoreType.DMA,
        pltpu.SemaphoreType.DMA,
        pltpu.SemaphoreType.DMA,
        pltpu.SemaphoreType.DMA,
        pltpu.SemaphoreType.DMA,
    ],
)
def sc_propagate(g_hbm, src_hbm, dst_hbm, out_hbm,
                 ib0, ib1, ib2, ib3, ib4, rb0, rb1, rb2, rb3, rb4, acc,
                 si0, si1, si2, si3, si4, sg0, sg1, sg2, sg3, sg4,
                 ss0, ss1, ss2, ss3, ss4, sb):
  c = lax.axis_index("c")
  s = lax.axis_index("s")
  ebase = (c * NS + s) * EPT
  row0 = s * RPT
  ibufs = (ib0, ib1, ib2, ib3, ib4)
  rbufs = (rb0, rb1, rb2, rb3, rb4)
  si = (si0, si1, si2, si3, si4)
  sg = (sg0, sg1, sg2, sg3, sg4)
  ss = (ss0, ss1, ss2, ss3, ss4)

  _zero_vmem_2d(rb0, CH, D)
  for k in range(RPT // CH):
    pltpu.async_copy(rb0, acc.at[pl.ds(row0 + k * CH, CH)], sb)
  for k in range(RPT // CH):
    pltpu.make_async_copy(rb0, acc.at[pl.ds(row0 + k * CH, CH)], sb).wait()
  plsc.subcore_barrier()

  def iissue(j, b):
    pltpu.async_copy(src_hbm.at[pl.ds(ebase + j * CH, CH)], ibufs[b].at[0],
                     si[b])
    pltpu.async_copy(dst_hbm.at[pl.ds(ebase + j * CH, CH)], ibufs[b].at[1],
                     si[b])

  def iwait(j, b):
    pltpu.make_async_copy(src_hbm.at[pl.ds(ebase + j * CH, CH)],
                          ibufs[b].at[0], si[b]).wait()
    pltpu.make_async_copy(dst_hbm.at[pl.ds(ebase + j * CH, CH)],
                          ibufs[b].at[1], si[b]).wait()

  def gissue(j, b):
    pltpu.async_copy(g_hbm.at[ibufs[b].at[0]], rbufs[b], sg[b])

  def gwait(j, b):
    pltpu.make_async_copy(g_hbm.at[ibufs[b].at[0]], rbufs[b], sg[b]).wait()

  def sissue(j, b):
    pltpu.async_copy(rbufs[b], acc.at[ibufs[b].at[1]], ss[b], add=True)

  def swait(j, b):
    pltpu.make_async_copy(rbufs[b], acc.at[ibufs[b].at[1]], ss[b]).wait()

  for b in range(3):
    iissue(b, b)
  for b in range(2):
    iwait(b, b)
    gissue(b, b)

  def outer(kk, carry):
    for b in range(NBUF):
      j = kk * NBUF + b
      nb3 = (b + 3) % NBUF
      nb2 = (b + 2) % NBUF
      @pl.when(j >= 2)
      def _ws():
        swait(j - 2, nb3)
      @pl.when(j + 3 < NCHUNK)
      def _ii():
        iissue(j + 3, nb3)
      @pl.when(j + 2 < NCHUNK)
      def _ig():
        iwait(j + 2, nb2)
        gissue(j + 2, nb2)
      gwait(j, b)
      sissue(j, b)
    return carry
  lax.fori_loop(0, NCHUNK // NBUF, outer, 0)
  for t in range(2):
    jj = NCHUNK - 2 + t
    swait(jj, jj % NBUF)

  plsc.subcore_barrier()
  for k in range(RPT // CH):
    r = row0 + k * CH
    pltpu.async_copy(acc.at[pl.ds(r, CH)], out_hbm.at[c, pl.ds(r, CH)], sb)
  for k in range(RPT // CH):
    r = row0 + k * CH
    pltpu.make_async_copy(acc.at[pl.ds(r, CH)], out_hbm.at[c, pl.ds(r, CH)],
                          sb).wait()


def _tc_first(x_ref, w_ref, degp_ref, g_ref, dinv_ref):
  deg2d = 1.0 + jnp.sum(degp_ref[...], axis=0)
  dinv2d = lax.rsqrt(deg2d)
  rowsel = (lax.broadcasted_iota(jnp.int32, (NP, HR), 1) ==
            lax.shift_right_logical(
                lax.broadcasted_iota(jnp.int32, (NP, HR), 0), 7)
            ).astype(jnp.float32)
  spread = jnp.dot(rowsel, dinv2d, preferred_element_type=jnp.float32)
  lanesel = (lax.broadcasted_iota(jnp.int32, (NP, D), 1) ==
             lax.bitwise_and(
                 lax.broadcasted_iota(jnp.int32, (NP, D), 0), 127))
  dcol = jnp.sum(jnp.where(lanesel, spread, 0.0), axis=1, keepdims=True)
  dinv = jnp.broadcast_to(dcol, (NP, D))
  dinv_ref[...] = dinv
  g_ref[...] = dinv * jnp.dot(x_ref[...], w_ref[...],
                              preferred_element_type=jnp.float32)


def _tc_mid(y_ref, g_ref, dinv_ref, b_ref, w_ref, gout_ref):
  dinv = dinv_ref[...]
  h = jnp.maximum(dinv * (y_ref[0] + y_ref[1] + g_ref[...]) + b_ref[...], 0.0)
  gout_ref[...] = dinv * jnp.dot(h, w_ref[...],
                                 preferred_element_type=jnp.float32)


def _tc_last(y_ref, g_ref, dinv_ref, b_ref, batch_ref, wl_ref, bl_ref,
             out_ref):
  dinv = dinv_ref[...]
  h = dinv * (y_ref[0] + y_ref[1] + g_ref[...]) + b_ref[...]
  ids = lax.broadcasted_iota(jnp.int32, (G, NP), 0)
  ind = (ids == jnp.broadcast_to(batch_ref[...], (G, NP))).astype(jnp.float32)
  sums = jnp.dot(ind, h, preferred_element_type=jnp.float32)
  cnts = jnp.sum(ind, axis=1, keepdims=True)
  pooled = sums / jnp.maximum(cnts, 1.0)
  logits = jnp.dot(pooled, wl_ref[...],
                   preferred_element_type=jnp.float32) + bl_ref[...]
  col = lax.broadcasted_iota(jnp.int32, (G, D), 1)
  lm = jnp.where(col < 10, logits, -jnp.inf)
  m = jnp.max(lm, axis=1, keepdims=True)
  lse = m + jnp.log(jnp.sum(jnp.exp(lm - m), axis=1, keepdims=True))
  out_ref[...] = logits - lse


def kernel(x, edge_index, batch, W1, b1, W2, b2, W3, b3, Wl, bl):
  src = edge_index[0].astype(jnp.int32)
  dst = edge_index[1].astype(jnp.int32)
  x = jnp.pad(x, ((0, NP - N), (0, 0)))
  batch2 = jnp.pad(batch.astype(jnp.int32), (0, NP - N),
                   constant_values=G).reshape(1, NP)
  wl_pad = jnp.zeros((D, D), jnp.float32).at[:, :10].set(Wl)
  bl_pad = jnp.zeros((1, D), jnp.float32).at[:, :10].set(bl)

  degp = sc_degree(dst)

  g1, dinv = pl.pallas_call(
      _tc_first,
      out_shape=(jax.ShapeDtypeStruct((NP, D), jnp.float32),
                 jax.ShapeDtypeStruct((NP, D), jnp.float32)),
  )(x, W1, degp)

  y1 = sc_propagate(g1, src, dst)

  mid = pl.pallas_call(
      _tc_mid, out_shape=jax.ShapeDtypeStruct((NP, D), jnp.float32))
  g2 = mid(y1, g1, dinv, b1.reshape(1, D), W2)
  y2 = sc_propagate(g2, src, dst)
  g3 = mid(y2, g2, dinv, b2.reshape(1, D), W3)
  y3 = sc_propagate(g3, src, dst)

  out = pl.pallas_call(
      _tc_last, out_shape=jax.ShapeDtypeStruct((G, D), jnp.float32))(
          y3, g3, dinv, b3.reshape(1, D), batch2, wl_pad, bl_pad)
  return out[:, :10]


# speedup vs baseline: 33.1401x; 1.0936x over previous
import functools

import jax
import jax.numpy as jnp
from jax import lax
from jax.experimental import pallas as pl
from jax.experimental.pallas import tpu as pltpu
from jax.experimental.pallas import tpu_sc as plsc

N = 10000
NP = 10240
E = 320000
D = 128
G = 64
NC, NS = 2, 16
NW = NC * NS
EPT = E // NW
CH = 40
NCHUNK = EPT // CH
RPT = NP // NS
ZR = 128
NBUF = 6
NITER = (NCHUNK + NBUF - 1) // NBUF

_mesh = plsc.VectorSubcoreMesh(
    core_axis_name="c", subcore_axis_name="s", num_cores=NC, num_subcores=NS)


def _zero_vmem_2d(ref, rows, cols):
  def row(i, _):
    for j in range(cols // 16):
      ref[i, pl.ds(j * 16, 16)] = jnp.zeros((16,), jnp.float32)
    return _
  lax.fori_loop(0, rows, row, 0)


HR = NP // 128


@functools.partial(
    pl.kernel,
    out_type=jax.ShapeDtypeStruct((NW, HR, 128), jnp.float32),
    mesh=_mesh,
    compiler_params=pltpu.CompilerParams(needs_layout_passes=False),
    scratch_types=[
        pltpu.VMEM((1, EPT), jnp.int32),
        pltpu.VMEM((HR, 128), jnp.float32),
        pltpu.SemaphoreType.DMA,
    ],
)
def sc_degree(dst_hbm, out_hbm, ib, hist, sb):
  c = lax.axis_index("c")
  s = lax.axis_index("s")
  wid = c * NS + s
  ebase = wid * EPT

  pltpu.async_copy(dst_hbm.at[pl.ds(ebase, EPT)], ib.at[0], sb)
  def z(i, u):
    for j8 in range(8):
      hist[i, pl.ds(j8 * 16, 16)] = jnp.zeros((16,), jnp.float32)
    return u
  lax.fori_loop(0, HR, z, 0)
  pltpu.make_async_copy(dst_hbm.at[pl.ds(ebase, EPT)], ib.at[0], sb).wait()

  ones = jnp.ones((16,), jnp.float32)
  def st(i, u):
    v = ib[0, pl.ds(i * 16, 16)]
    plsc.addupdate_scatter(
        hist, [lax.shift_right_logical(v, 7), lax.bitwise_and(v, 127)], ones)
    return u
  lax.fori_loop(0, EPT // 16, st, 0)
  pltpu.sync_copy(hist, out_hbm.at[wid])


@functools.partial(
    pl.kernel,
    out_type=jax.ShapeDtypeStruct((NC, NP, D), jnp.float32),
    mesh=_mesh,
    scratch_types=[
        pltpu.VMEM((2, CH), jnp.int32),
        pltpu.VMEM((2, CH), jnp.int32),
        pltpu.VMEM((2, CH), jnp.int32),
        pltpu.VMEM((2, CH), jnp.int32),
        pltpu.VMEM((2, CH), jnp.int32),
        pltpu.VMEM((2, CH), jnp.int32),
        pltpu.VMEM((CH, D), jnp.float32),
        pltpu.VMEM((CH, D), jnp.float32),
        pltpu.VMEM((CH, D), jnp.float32),
        pltpu.VMEM((CH, D), jnp.float32),
        pltpu.VMEM((CH, D), jnp.float32),
        pltpu.VMEM((CH, D), jnp.float32),
        pltpu.VMEM_SHARED((NP, D), jnp.float32),
        pltpu.SemaphoreType.DMA,
        pltpu.SemaphoreType.DMA,
        pltpu.SemaphoreType.DMA,
        pltpu.SemaphoreType.DMA,
        pltpu.SemaphoreType.DMA,
        pltpu.SemaphoreType.DMA,
        pltpu.SemaphoreType.DMA,
        pltpu.SemaphoreType.DMA,
        pltpu.SemaphoreType.DMA,
        pltpu.SemaphoreType.DMA,
        pltpu.SemaphoreType.DMA,
        pltpu.SemaphoreType.DMA,
        pltpu.SemaphoreType.DMA,
        pltpu.SemaphoreType.DMA,
        pltpu.SemaphoreType.DMA,
        pltpu.SemaphoreType.DMA,
        pltpu.SemaphoreType.DMA,
        pltpu.SemaphoreType.DMA,
        pltpu.SemaphoreType.DMA,
    ],
)
def sc_propagate(g_hbm, src_hbm, dst_hbm, out_hbm,
                 ib0, ib1, ib2, ib3, ib4, ib5, rb0, rb1, rb2, rb3, rb4, rb5,
                 acc, si0, si1, si2, si3, si4, si5,
                 sg0, sg1, sg2, sg3, sg4, sg5,
                 ss0, ss1, ss2, ss3, ss4, ss5, sb):
  c = lax.axis_index("c")
  s = lax.axis_index("s")
  ebase = (c * NS + s) * EPT
  row0 = s * RPT
  ibufs = (ib0, ib1, ib2, ib3, ib4, ib5)
  rbufs = (rb0, rb1, rb2, rb3, rb4, rb5)
  si = (si0, si1, si2, si3, si4, si5)
  sg = (sg0, sg1, sg2, sg3, sg4, sg5)
  ss = (ss0, ss1, ss2, ss3, ss4, ss5)

  def iissue(j, b):
    pltpu.async_copy(src_hbm.at[pl.ds(ebase + j * CH, CH)], ibufs[b].at[0],
                     si[b])
    pltpu.async_copy(dst_hbm.at[pl.ds(ebase + j * CH, CH)], ibufs[b].at[1],
                     si[b])

  def iwait(j, b):
    pltpu.make_async_copy(src_hbm.at[pl.ds(ebase + j * CH, CH)],
                          ibufs[b].at[0], si[b]).wait()
    pltpu.make_async_copy(dst_hbm.at[pl.ds(ebase + j * CH, CH)],
                          ibufs[b].at[1], si[b]).wait()

  def gissue(j, b):
    pltpu.async_copy(g_hbm.at[ibufs[b].at[0]], rbufs[b], sg[b])

  def gwait(j, b):
    pltpu.make_async_copy(g_hbm.at[ibufs[b].at[0]], rbufs[b], sg[b]).wait()

  def sissue(j, b):
    pltpu.async_copy(rbufs[b], acc.at[ibufs[b].at[1]], ss[b], add=True)

  def swait(j, b):
    pltpu.make_async_copy(rbufs[b], acc.at[ibufs[b].at[1]], ss[b]).wait()

  for b in range(4):
    iissue(b, b)

  _zero_vmem_2d(rb5, CH, D)
  for k in range(RPT // CH):
    pltpu.async_copy(rb5, acc.at[pl.ds(row0 + k * CH, CH)], sb)
  for k in range(RPT // CH):
    pltpu.make_async_copy(rb5, acc.at[pl.ds(row0 + k * CH, CH)], sb).wait()

  for b in range(3):
    iwait(b, b)
    gissue(b, b)
  plsc.subcore_barrier()

  def outer(kk, carry):
    for b in range(NBUF):
      j = kk * NBUF + b
      nb4 = (b + 4) % NBUF
      nb3 = (b + 3) % NBUF
      @pl.when(j >= 2)
      def _ws():
        swait(j - 2, nb4)
      @pl.when(j + 4 < NCHUNK)
      def _ii():
        iissue(j + 4, nb4)
      @pl.when(j + 3 < NCHUNK)
      def _ig():
        iwait(j + 3, nb3)
        gissue(j + 3, nb3)
      @pl.when(j < NCHUNK)
      def _gs():
        gwait(j, b)
        sissue(j, b)
    return carry
  lax.fori_loop(0, NITER, outer, 0)

  plsc.subcore_barrier()
  for k in range(RPT // CH):
    r = row0 + k * CH
    pltpu.async_copy(acc.at[pl.ds(r, CH)], out_hbm.at[c, pl.ds(r, CH)], sb)
  for k in range(RPT // CH):
    r = row0 + k * CH
    pltpu.make_async_copy(acc.at[pl.ds(r, CH)], out_hbm.at[c, pl.ds(r, CH)],
                          sb).wait()


def _tc_first(x_ref, w_ref, degp_ref, g_ref, dinv_ref):
  deg2d = 1.0 + jnp.sum(degp_ref[...], axis=0)
  dinv2d = lax.rsqrt(deg2d)
  rowsel = (lax.broadcasted_iota(jnp.int32, (NP, HR), 1) ==
            lax.shift_right_logical(
                lax.broadcasted_iota(jnp.int32, (NP, HR), 0), 7)
            ).astype(jnp.float32)
  spread = jnp.dot(rowsel, dinv2d, preferred_element_type=jnp.float32)
  lanesel = (lax.broadcasted_iota(jnp.int32, (NP, D), 1) ==
             lax.bitwise_and(
                 lax.broadcasted_iota(jnp.int32, (NP, D), 0), 127))
  dcol = jnp.sum(jnp.where(lanesel, spread, 0.0), axis=1, keepdims=True)
  dinv = jnp.broadcast_to(dcol, (NP, D))
  dinv_ref[...] = dinv
  g_ref[...] = dinv * jnp.dot(x_ref[...], w_ref[...],
                              preferred_element_type=jnp.float32)


def _tc_mid(y_ref, g_ref, dinv_ref, b_ref, w_ref, gout_ref):
  dinv = dinv_ref[...]
  h = jnp.maximum(dinv * (y_ref[0] + y_ref[1] + g_ref[...]) + b_ref[...], 0.0)
  gout_ref[...] = dinv * jnp.dot(h, w_ref[...],
                                 preferred_element_type=jnp.float32)


def _tc_last(y_ref, g_ref, dinv_ref, b_ref, batch_ref, wl_ref, bl_ref,
             out_ref):
  dinv = dinv_ref[...]
  h = dinv * (y_ref[0] + y_ref[1] + g_ref[...]) + b_ref[...]
  ids = lax.broadcasted_iota(jnp.int32, (G, NP), 0)
  ind = (ids == jnp.broadcast_to(batch_ref[...], (G, NP))).astype(jnp.float32)
  sums = jnp.dot(ind, h, preferred_element_type=jnp.float32)
  cnts = jnp.sum(ind, axis=1, keepdims=True)
  pooled = sums / jnp.maximum(cnts, 1.0)
  logits = jnp.dot(pooled, wl_ref[...],
                   preferred_element_type=jnp.float32) + bl_ref[...]
  col = lax.broadcasted_iota(jnp.int32, (G, D), 1)
  lm = jnp.where(col < 10, logits, -jnp.inf)
  m = jnp.max(lm, axis=1, keepdims=True)
  lse = m + jnp.log(jnp.sum(jnp.exp(lm - m), axis=1, keepdims=True))
  out_ref[...] = logits - lse


def kernel(x, edge_index, batch, W1, b1, W2, b2, W3, b3, Wl, bl):
  src = edge_index[0].astype(jnp.int32)
  dst = edge_index[1].astype(jnp.int32)
  x = jnp.pad(x, ((0, NP - N), (0, 0)))
  batch2 = jnp.pad(batch.astype(jnp.int32), (0, NP - N),
                   constant_values=G).reshape(1, NP)
  wl_pad = jnp.zeros((D, D), jnp.float32).at[:, :10].set(Wl)
  bl_pad = jnp.zeros((1, D), jnp.float32).at[:, :10].set(bl)

  degp = sc_degree(dst)

  g1, dinv = pl.pallas_call(
      _tc_first,
      out_shape=(jax.ShapeDtypeStruct((NP, D), jnp.float32),
                 jax.ShapeDtypeStruct((NP, D), jnp.float32)),
  )(x, W1, degp)

  y1 = sc_propagate(g1, src, dst)

  mid = pl.pallas_call(
      _tc_mid, out_shape=jax.ShapeDtypeStruct((NP, D), jnp.float32))
  g2 = mid(y1, g1, dinv, b1.reshape(1, D), W2)
  y2 = sc_propagate(g2, src, dst)
  g3 = mid(y2, g2, dinv, b2.reshape(1, D), W3)
  y3 = sc_propagate(g3, src, dst)

  out = pl.pallas_call(
      _tc_last, out_shape=jax.ShapeDtypeStruct((G, D), jnp.float32))(
          y3, g3, dinv, b3.reshape(1, D), batch2, wl_pad, bl_pad)
  return out[:, :10]
